# baseline (speedup 1.0000x reference)
import numpy as np

import jax
import jax.numpy as jnp
from jax import lax
from jax.experimental import pallas as pl
from jax.experimental.pallas import tpu as pltpu
from jax.experimental.pallas import tpu_sc as plsc

N = 32768
NT = 16
C = N // NT
NV = C // 16
RADIX = 256
NG = RADIX // 16
SHIFTS = (0, 8, 16, 24)

_MIN32 = np.int32(-2147483648)


def _flip(v):
    m = lax.shift_right_arithmetic(v, 31)
    return lax.bitwise_xor(v, lax.bitwise_or(m, _MIN32))


def _digit(k, shift):
    return lax.bitwise_and(lax.shift_right_logical(k, shift), np.int32(255))


def _transpose_addr(j):
    return lax.bitwise_or(
        lax.shift_left(lax.bitwise_and(j, np.int32(127)), 4),
        lax.shift_right_logical(j, 7))


def _sort_body(times_h, labels_h, out_t_h, out_l_h,
               key_v, idx_v, hist_v, rank_v, tt_v, allh_v, gtot_v, tpre_v,
               gt_v, pfx_v, carry_v, dest_v, tbuf_v, lbuf_v, idx2_v, drain_v,
               ka, kb, ia, ib, t_sh, l_sh, hist_sh, sem, sem2):
    wid = lax.axis_index("sub")
    base = wid * C
    lane = lax.iota(jnp.int32, 16)
    lane256 = lane * 256
    ones = jnp.ones((16,), jnp.int32)
    zeros = jnp.zeros((16,), jnp.int32)
    bsl = pl.ds(base, C)

    def zero_hist():
        @pl.loop(0, RADIX)
        def _(i):
            hist_v[pl.ds(i * 16, 16)] = zeros

    def ingest():
        pltpu.sync_copy(times_h.at[bsl], tbuf_v)
        cp_t = pltpu.async_copy(times_h.at[bsl], t_sh.at[bsl], sem)
        cp_l = pltpu.async_copy(labels_h.at[bsl], l_sh.at[bsl], sem)
        zero_hist()

        @pl.loop(0, NV)
        def _(i):
            pos = i * 16 + lane
            x = tbuf_v[pl.ds(i * 16, 16)] + np.float32(0.0)
            k = _flip(lax.bitcast_convert_type(x, jnp.int32))
            ta = _transpose_addr(pos)
            plsc.store_scatter(key_v, [ta], k)
            plsc.store_scatter(idx_v, [ta], base + pos)

        cp_t.wait()
        cp_l.wait()

    def drain(n):
        for _ in range(n):
            pltpu.make_async_copy(times_h.at[pl.ds(0, 128)], drain_v, sem).wait()

    def pass_a(p, src_k, src_i):
        shift = SHIFTS[p]
        cp_i = None
        if p > 0:
            cp_k = pltpu.async_copy(src_k.at[bsl], key_v, sem2)
            cp_i = pltpu.async_copy(src_i.at[bsl], idx_v, sem2)
            cp_k.wait()

        @pl.loop(0, NV)
        def _(i):
            k = key_v[pl.ds(i * 16, 16)]
            plsc.addupdate_scatter(hist_v, [lane256 + _digit(k, shift)], ones)

        @pl.loop(0, NG)
        def _(g):
            acc = zeros
            for l in range(16):
                off = l * 256 + g * 16
                rank_v[pl.ds(off, 16)] = acc
                acc = acc + hist_v[pl.ds(off, 16)]
            tt_v[pl.ds(g * 16, 16)] = acc

        pltpu.sync_copy(tt_v, hist_sh.at[wid])
        return cp_i

    def pass_c(p, dst_k, dst_i, write_keys, cp_i):
        shift = SHIFTS[p]
        pltpu.sync_copy(hist_sh, allh_v)

        @pl.loop(0, NG)
        def _(g):
            acc = zeros
            tp = zeros
            for t in range(NT):
                v = allh_v[t, pl.ds(g * 16, 16)]
                tp = tp + jnp.where(t < wid, v, 0)
                acc = acc + v
            gtot_v[pl.ds(g * 16, 16)] = acc
            tpre_v[pl.ds(g * 16, 16)] = tp

        @pl.loop(0, NG)
        def _(g):
            pfx_v[pl.ds(g * 16, 16)] = plsc.cumsum(gtot_v[pl.ds(g * 16, 16)])
        tot = plsc.load_gather(pfx_v, [lane * 16 + 15])
        carry_v[...] = plsc.cumsum(tot) - tot

        @pl.loop(0, NG)
        def _(g):
            sl16 = pl.ds(g * 16, 16)
            gv = jnp.full((16,), g, jnp.int32)
            gt_v[sl16] = (pfx_v[sl16] - gtot_v[sl16] + tpre_v[sl16]
                          + plsc.load_gather(carry_v, [gv]))

        if cp_i is not None:
            cp_i.wait()

        @pl.loop(0, NV // 8)
        def _(j):
            for q in range(8):
                i = j * 8 + q
                k = key_v[pl.ds(i * 16, 16)]
                d = _digit(k, shift)
                addr = lane256 + d
                r = plsc.load_gather(rank_v, [addr])
                plsc.addupdate_scatter(rank_v, [addr], ones)
                r = r + plsc.load_gather(gt_v, [d])
                p_t = lax.bitwise_or(
                    lax.bitwise_and(r, np.int32(~2047)),
                    _transpose_addr(lax.bitwise_and(r, np.int32(2047))))
                dest_v[j, pl.ds(q * 16, 16)] = p_t
            sl = pl.ds(j * 128, 128)
            if write_keys:
                pltpu.async_copy(key_v.at[sl], dst_k.at[dest_v.at[j]], sem)
            pltpu.async_copy(idx_v.at[sl], dst_i.at[dest_v.at[j]], sem)

        zero_hist()
        drain(32 if write_keys else 16)

    def final_out():
        pltpu.sync_copy(ia.at[bsl], idx_v)

        @pl.loop(0, 16)
        def _(j):
            for q in range(8):
                addr = (q * 16 + lane) * 16 + j
                x = plsc.load_gather(idx_v, [addr])
                idx2_v[pl.ds(j * 128 + q * 16, 16)] = x
            sl = pl.ds(j * 128, 128)
            pltpu.async_copy(t_sh.at[idx2_v.at[sl]], tbuf_v.at[sl], sem)
            pltpu.async_copy(l_sh.at[idx2_v.at[sl]], lbuf_v.at[sl], sem)

        drain(32)
        pltpu.sync_copy(tbuf_v, out_t_h.at[bsl])
        pltpu.sync_copy(lbuf_v, out_l_h.at[bsl])

    sched = [
        (None, None, kb, ib, True),
        (kb, ib, ka, ia, True),
        (ka, ia, kb, ib, True),
        (kb, ib, ka, ia, False),
    ]

    for p, (sk, si, dk, di, wk) in enumerate(sched):
        if p == 0:
            ingest()
        cp_i = pass_a(p, sk, si)
        plsc.subcore_barrier()
        pass_c(p, dk, di, wk, cp_i)
        plsc.subcore_barrier()

    final_out()


def _build():
    mesh = plsc.VectorSubcoreMesh(
        core_axis_name="core", subcore_axis_name="sub", num_cores=1)
    return pl.kernel(
        _sort_body,
        out_type=(jax.ShapeDtypeStruct((N,), jnp.float32),
                  jax.ShapeDtypeStruct((N,), jnp.int32)),
        mesh=mesh,
        compiler_params=pltpu.CompilerParams(needs_layout_passes=False),
        scratch_types=[
            pltpu.VMEM((C,), jnp.int32),
            pltpu.VMEM((C,), jnp.int32),
            pltpu.VMEM((RADIX * 16,), jnp.int32),
            pltpu.VMEM((RADIX * 16,), jnp.int32),
            pltpu.VMEM((RADIX,), jnp.int32),
            pltpu.VMEM((NT, RADIX), jnp.int32),
            pltpu.VMEM((RADIX,), jnp.int32),
            pltpu.VMEM((RADIX,), jnp.int32),
            pltpu.VMEM((RADIX,), jnp.int32),
            pltpu.VMEM((RADIX,), jnp.int32),
            pltpu.VMEM((16,), jnp.int32),
            pltpu.VMEM((16, 128), jnp.int32),
            pltpu.VMEM((C,), jnp.float32),
            pltpu.VMEM((C,), jnp.int32),
            pltpu.VMEM((C,), jnp.int32),
            pltpu.VMEM((128,), jnp.int32),
            pltpu.VMEM_SHARED((N,), jnp.int32),
            pltpu.VMEM_SHARED((N,), jnp.int32),
            pltpu.VMEM_SHARED((N,), jnp.int32),
            pltpu.VMEM_SHARED((N,), jnp.int32),
            pltpu.VMEM_SHARED((N,), jnp.float32),
            pltpu.VMEM_SHARED((N,), jnp.int32),
            pltpu.VMEM_SHARED((NT, RADIX), jnp.int32),
            pltpu.SemaphoreType.DMA,
            pltpu.SemaphoreType.DMA,
        ],
    )


def kernel(times, labels):
    labels32 = labels.astype(jnp.int32)
    out_t, out_l = _build()(times, labels32)
    return out_t, out_l.astype(labels.dtype)


# speedup vs baseline: 1.0420x; 1.0420x over previous
import numpy as np

import jax
import jax.numpy as jnp
from jax import lax
from jax.experimental import pallas as pl
from jax.experimental.pallas import tpu as pltpu
from jax.experimental.pallas import tpu_sc as plsc

N = 32768
NT = 16
C = N // NT
NV = C // 16
RADIX = 256
NG = RADIX // 16
SHIFTS = (0, 8, 16, 24)

_MIN32 = np.int32(-2147483648)


def _flip(v):
    m = lax.shift_right_arithmetic(v, 31)
    return lax.bitwise_xor(v, lax.bitwise_or(m, _MIN32))


def _digit(k, shift):
    return lax.bitwise_and(lax.shift_right_logical(k, shift), np.int32(255))


def _sort_body(times_h, labels_h, out_t_h, out_l_h,
               key_v, idx_v, hist_v, tab_v, allh_v, gtot_v, tpre_v,
               pfx_v, carry_v, dest_v, tbuf_v, lbuf_v, drain_v,
               ka, kb, ia, ib, t_sh, l_sh, hist_sh, sem, sem2):
    wid = lax.axis_index("sub")
    base = wid * C
    lane = lax.iota(jnp.int32, 16)
    ones = jnp.ones((16,), jnp.int32)
    zeros = jnp.zeros((16,), jnp.int32)
    bsl = pl.ds(base, C)

    def zero_hist():
        @pl.loop(0, NG)
        def _(i):
            hist_v[pl.ds(i * 16, 16)] = zeros

    def ingest():
        pltpu.sync_copy(times_h.at[bsl], tbuf_v)
        cp_t = pltpu.async_copy(times_h.at[bsl], t_sh.at[bsl], sem)
        cp_l = pltpu.async_copy(labels_h.at[bsl], l_sh.at[bsl], sem)
        zero_hist()

        @pl.loop(0, NV)
        def _(i):
            x = tbuf_v[pl.ds(i * 16, 16)] + np.float32(0.0)
            k = _flip(lax.bitcast_convert_type(x, jnp.int32))
            key_v[pl.ds(i * 16, 16)] = k
            idx_v[pl.ds(i * 16, 16)] = base + i * 16 + lane

        cp_t.wait()
        cp_l.wait()

    def drain(n):
        for _ in range(n):
            pltpu.make_async_copy(times_h.at[pl.ds(0, 128)], drain_v, sem).wait()

    def pass_a(p, src_k, src_i):
        shift = SHIFTS[p]
        cp_i = None
        if p > 0:
            cp_k = pltpu.async_copy(src_k.at[bsl], key_v, sem2)
            cp_i = pltpu.async_copy(src_i.at[bsl], idx_v, sem2)
            cp_k.wait()

        @pl.loop(0, NV)
        def _(i):
            k = key_v[pl.ds(i * 16, 16)]
            d = _digit(k, shift)
            cnt, last = plsc.scan_count(d)
            plsc.addupdate_scatter(hist_v, [d], cnt, mask=last)

        pltpu.sync_copy(hist_v, hist_sh.at[wid])
        return cp_i

    def pass_c(p, dst_k, dst_i, write_keys, cp_i):
        shift = SHIFTS[p]
        pltpu.sync_copy(hist_sh, allh_v)

        @pl.loop(0, NG)
        def _(g):
            acc = zeros
            tp = zeros
            for t in range(NT):
                v = allh_v[t, pl.ds(g * 16, 16)]
                tp = tp + jnp.where(t < wid, v, 0)
                acc = acc + v
            gtot_v[pl.ds(g * 16, 16)] = acc
            tpre_v[pl.ds(g * 16, 16)] = tp

        @pl.loop(0, NG)
        def _(g):
            pfx_v[pl.ds(g * 16, 16)] = plsc.cumsum(gtot_v[pl.ds(g * 16, 16)])
        tot = plsc.load_gather(pfx_v, [lane * 16 + 15])
        carry_v[...] = plsc.cumsum(tot) - tot

        @pl.loop(0, NG)
        def _(g):
            sl16 = pl.ds(g * 16, 16)
            gv = jnp.full((16,), g, jnp.int32)
            tab_v[sl16] = (pfx_v[sl16] - gtot_v[sl16] + tpre_v[sl16]
                           + plsc.load_gather(carry_v, [gv]))

        if cp_i is not None:
            cp_i.wait()

        @pl.loop(0, NV // 8)
        def _(j):
            for q in range(8):
                i = j * 8 + q
                k = key_v[pl.ds(i * 16, 16)]
                d = _digit(k, shift)
                cnt, last = plsc.scan_count(d)
                r = plsc.load_gather(tab_v, [d]) + cnt - ones
                plsc.addupdate_scatter(tab_v, [d], cnt, mask=last)
                dest_v[j, pl.ds(q * 16, 16)] = r
            sl = pl.ds(j * 128, 128)
            if write_keys:
                pltpu.async_copy(key_v.at[sl], dst_k.at[dest_v.at[j]], sem)
            pltpu.async_copy(idx_v.at[sl], dst_i.at[dest_v.at[j]], sem)

        zero_hist()
        drain(32 if write_keys else 16)

    def final_out():
        pltpu.sync_copy(ia.at[bsl], idx_v)
        for j in range(16):
            sl = pl.ds(j * 128, 128)
            pltpu.async_copy(t_sh.at[idx_v.at[sl]], tbuf_v.at[sl], sem)
            pltpu.async_copy(l_sh.at[idx_v.at[sl]], lbuf_v.at[sl], sem)
        drain(32)
        pltpu.sync_copy(tbuf_v, out_t_h.at[bsl])
        pltpu.sync_copy(lbuf_v, out_l_h.at[bsl])

    sched = [
        (None, None, kb, ib, True),
        (kb, ib, ka, ia, True),
        (ka, ia, kb, ib, True),
        (kb, ib, ka, ia, False),
    ]

    for p, (sk, si, dk, di, wk) in enumerate(sched):
        if p == 0:
            ingest()
        cp_i = pass_a(p, sk, si)
        plsc.subcore_barrier()
        pass_c(p, dk, di, wk, cp_i)
        plsc.subcore_barrier()

    final_out()


def _build():
    mesh = plsc.VectorSubcoreMesh(
        core_axis_name="core", subcore_axis_name="sub", num_cores=1)
    return pl.kernel(
        _sort_body,
        out_type=(jax.ShapeDtypeStruct((N,), jnp.float32),
                  jax.ShapeDtypeStruct((N,), jnp.int32)),
        mesh=mesh,
        compiler_params=pltpu.CompilerParams(needs_layout_passes=False),
        scratch_types=[
            pltpu.VMEM((C,), jnp.int32),
            pltpu.VMEM((C,), jnp.int32),
            pltpu.VMEM((RADIX,), jnp.int32),
            pltpu.VMEM((RADIX,), jnp.int32),
            pltpu.VMEM((NT, RADIX), jnp.int32),
            pltpu.VMEM((RADIX,), jnp.int32),
            pltpu.VMEM((RADIX,), jnp.int32),
            pltpu.VMEM((RADIX,), jnp.int32),
            pltpu.VMEM((16,), jnp.int32),
            pltpu.VMEM((16, 128), jnp.int32),
            pltpu.VMEM((C,), jnp.float32),
            pltpu.VMEM((C,), jnp.int32),
            pltpu.VMEM((128,), jnp.int32),
            pltpu.VMEM_SHARED((N,), jnp.int32),
            pltpu.VMEM_SHARED((N,), jnp.int32),
            pltpu.VMEM_SHARED((N,), jnp.int32),
            pltpu.VMEM_SHARED((N,), jnp.int32),
            pltpu.VMEM_SHARED((N,), jnp.float32),
            pltpu.VMEM_SHARED((N,), jnp.int32),
            pltpu.VMEM_SHARED((NT, RADIX), jnp.int32),
            pltpu.SemaphoreType.DMA,
            pltpu.SemaphoreType.DMA,
        ],
    )


def kernel(times, labels):
    labels32 = labels.astype(jnp.int32)
    out_t, out_l = _build()(times, labels32)
    return out_t, out_l.astype(labels.dtype)


# speedup vs baseline: 1.0459x; 1.0037x over previous
import numpy as np

import jax
import jax.numpy as jnp
from jax import lax
from jax.experimental import pallas as pl
from jax.experimental.pallas import tpu as pltpu
from jax.experimental.pallas import tpu_sc as plsc

N = 32768
NT = 16
C = N // NT
NV = C // 16
RADIX = 256
NG = RADIX // 16
SHIFTS = (0, 8, 16, 24)

_MIN32 = np.int32(-2147483648)


def _flip(v):
    m = lax.shift_right_arithmetic(v, 31)
    return lax.bitwise_xor(v, lax.bitwise_or(m, _MIN32))


def _digit(k, shift):
    return lax.bitwise_and(lax.shift_right_logical(k, shift), np.int32(255))


def _sort_body(times_h, labels_h, out_t_h, out_l_h,
               key_v, idx_v, hist_v, tab_v, allh_v, gtot_v, tpre_v,
               pfx_v, carry_v, dest_v, tbuf_v, lbuf_v, wv, drain_v,
               ka, kb, ia, ib, t_sh, l_sh, hist_sh, sem, sem2):
    wid = lax.axis_index("sub")
    base = wid * C
    lane = lax.iota(jnp.int32, 16)
    ones = jnp.ones((16,), jnp.int32)
    zeros = jnp.zeros((16,), jnp.int32)
    bsl = pl.ds(base, C)

    def zero_hist():
        @pl.loop(0, NG)
        def _(i):
            hist_v[pl.ds(i * 16, 16)] = zeros

    def ingest():
        pltpu.sync_copy(times_h.at[bsl], tbuf_v)
        cp_t = pltpu.async_copy(times_h.at[bsl], t_sh.at[bsl], sem)
        cp_l = pltpu.async_copy(labels_h.at[bsl], l_sh.at[bsl], sem)
        zero_hist()

        @pl.loop(0, NV)
        def _(i):
            x = tbuf_v[pl.ds(i * 16, 16)] + np.float32(0.0)
            k = _flip(lax.bitcast_convert_type(x, jnp.int32))
            key_v[pl.ds(i * 16, 16)] = k
            idx_v[pl.ds(i * 16, 16)] = base + i * 16 + lane

        cp_t.wait()
        cp_l.wait()

    def drain(n):
        for _ in range(n):
            pltpu.make_async_copy(times_h.at[pl.ds(0, 128)], drain_v, sem).wait()

    def pass_a(p, src_k, src_i):
        shift = SHIFTS[p]
        cp_i = None
        if p == 1:
            cp_k = pltpu.async_copy(src_k.at[bsl], key_v, sem2)
            cp_i = pltpu.async_copy(src_i.at[bsl], idx_v, sem2)
            cp_k.wait()
        elif p > 1:
            pltpu.sync_copy(src_k.at[bsl], key_v)

        @pl.loop(0, NV)
        def _(i):
            k = key_v[pl.ds(i * 16, 16)]
            d = _digit(k, shift)
            cnt, last = plsc.scan_count(d)
            plsc.addupdate_scatter(hist_v, [d], cnt, mask=last)

        pltpu.sync_copy(hist_v, hist_sh.at[wid])
        return cp_i

    def pass_c(p, dst_k, dst_i, write_keys, cp_i):
        shift = SHIFTS[p]
        pltpu.sync_copy(hist_sh, allh_v)

        @pl.loop(0, NG)
        def _(g):
            acc = zeros
            tp = zeros
            for t in range(NT):
                v = allh_v[t, pl.ds(g * 16, 16)]
                tp = tp + jnp.where(t < wid, v, 0)
                acc = acc + v
            gtot_v[pl.ds(g * 16, 16)] = acc
            tpre_v[pl.ds(g * 16, 16)] = tp

        @pl.loop(0, NG)
        def _(g):
            pfx_v[pl.ds(g * 16, 16)] = plsc.cumsum(gtot_v[pl.ds(g * 16, 16)])
        tot = plsc.load_gather(pfx_v, [lane * 16 + 15])
        carry_v[...] = plsc.cumsum(tot) - tot

        @pl.loop(0, NG)
        def _(g):
            sl16 = pl.ds(g * 16, 16)
            gv = jnp.full((16,), g, jnp.int32)
            tab_v[sl16] = (pfx_v[sl16] - gtot_v[sl16] + tpre_v[sl16]
                           + plsc.load_gather(carry_v, [gv]))

        if cp_i is not None:
            cp_i.wait()

        @pl.loop(0, NV // 8)
        def _(j):
            for q in range(8):
                i = j * 8 + q
                sl16 = pl.ds(i * 16, 16)
                k = key_v[sl16]
                d = _digit(k, shift)
                cnt, last = plsc.scan_count(d)
                r = plsc.load_gather(tab_v, [d]) + cnt - ones
                plsc.addupdate_scatter(tab_v, [d], cnt, mask=last)
                dest_v[j, pl.ds(q * 16, 16)] = r
                if p == 1:
                    w = lax.bitwise_or(
                        lax.bitwise_and(k, np.int32(-65536)), idx_v[sl16])
                    wv[sl16] = w
                elif p == 3:
                    wv[sl16] = lax.bitwise_and(k, np.int32(0x7FFF))
            sl = pl.ds(j * 128, 128)
            if p == 0:
                pltpu.async_copy(key_v.at[sl], dst_k.at[dest_v.at[j]], sem)
                pltpu.async_copy(idx_v.at[sl], dst_i.at[dest_v.at[j]], sem)
            elif p == 2:
                pltpu.async_copy(key_v.at[sl], dst_k.at[dest_v.at[j]], sem)
            else:
                pltpu.async_copy(wv.at[sl], dst_k.at[dest_v.at[j]], sem)

        zero_hist()
        drain(32 if p == 0 else 16)

    def final_out():
        pltpu.sync_copy(ia.at[bsl], idx_v)
        for j in range(16):
            sl = pl.ds(j * 128, 128)
            pltpu.async_copy(t_sh.at[idx_v.at[sl]], tbuf_v.at[sl], sem)
            pltpu.async_copy(l_sh.at[idx_v.at[sl]], lbuf_v.at[sl], sem)
        drain(32)
        pltpu.sync_copy(tbuf_v, out_t_h.at[bsl])
        pltpu.sync_copy(lbuf_v, out_l_h.at[bsl])

    sched = [
        (None, None, kb, ib, True),
        (kb, ib, ka, None, True),
        (ka, None, kb, None, True),
        (kb, None, ia, None, False),
    ]

    for p, (sk, si, dk, di, wk) in enumerate(sched):
        if p == 0:
            ingest()
        cp_i = pass_a(p, sk, si)
        plsc.subcore_barrier()
        pass_c(p, dk, di, wk, cp_i)
        plsc.subcore_barrier()

    final_out()


def _build():
    mesh = plsc.VectorSubcoreMesh(
        core_axis_name="core", subcore_axis_name="sub", num_cores=1)
    return pl.kernel(
        _sort_body,
        out_type=(jax.ShapeDtypeStruct((N,), jnp.float32),
                  jax.ShapeDtypeStruct((N,), jnp.int32)),
        mesh=mesh,
        compiler_params=pltpu.CompilerParams(needs_layout_passes=False),
        scratch_types=[
            pltpu.VMEM((C,), jnp.int32),
            pltpu.VMEM((C,), jnp.int32),
            pltpu.VMEM((RADIX,), jnp.int32),
            pltpu.VMEM((RADIX,), jnp.int32),
            pltpu.VMEM((NT, RADIX), jnp.int32),
            pltpu.VMEM((RADIX,), jnp.int32),
            pltpu.VMEM((RADIX,), jnp.int32),
            pltpu.VMEM((RADIX,), jnp.int32),
            pltpu.VMEM((16,), jnp.int32),
            pltpu.VMEM((16, 128), jnp.int32),
            pltpu.VMEM((C,), jnp.float32),
            pltpu.VMEM((C,), jnp.int32),
            pltpu.VMEM((C,), jnp.int32),
            pltpu.VMEM((128,), jnp.int32),
            pltpu.VMEM_SHARED((N,), jnp.int32),
            pltpu.VMEM_SHARED((N,), jnp.int32),
            pltpu.VMEM_SHARED((N,), jnp.int32),
            pltpu.VMEM_SHARED((N,), jnp.int32),
            pltpu.VMEM_SHARED((N,), jnp.float32),
            pltpu.VMEM_SHARED((N,), jnp.int32),
            pltpu.VMEM_SHARED((NT, RADIX), jnp.int32),
            pltpu.SemaphoreType.DMA,
            pltpu.SemaphoreType.DMA,
        ],
    )


def kernel(times, labels):
    labels32 = labels.astype(jnp.int32)
    out_t, out_l = _build()(times, labels32)
    return out_t, out_l.astype(labels.dtype)


# speedup vs baseline: 1.1052x; 1.0567x over previous
import numpy as np

import jax
import jax.numpy as jnp
from jax import lax
from jax.experimental import pallas as pl
from jax.experimental.pallas import tpu as pltpu
from jax.experimental.pallas import tpu_sc as plsc

N = 32768
NT = 16
C = N // NT
NV = C // 16
RADIX = 256
NG = RADIX // 16
SHIFTS = (0, 8, 16, 24)

_MIN32 = np.int32(-2147483648)


def _flip(v):
    m = lax.shift_right_arithmetic(v, 31)
    return lax.bitwise_xor(v, lax.bitwise_or(m, _MIN32))


def _digit(k, shift):
    return lax.bitwise_and(lax.shift_right_logical(k, shift), np.int32(255))


def _sort_body(times_h, labels_h, out_t_h, out_l_h,
               key_v, idx_v, hist_v, tab_v, allh_v, gtot_v, tpre_v,
               pfx_v, carry_v, dest_v, tbuf_v, lbuf_v, wv, drain_v,
               ones128_v, zero256_v, addr2_v,
               ka, kb, ia, ib, t_sh, l_sh, hist_sh, h21_sh, h22_sh, h23_sh,
               sem, sem2):
    wid = lax.axis_index("sub")
    base = wid * C
    lane = lax.iota(jnp.int32, 16)
    ones = jnp.ones((16,), jnp.int32)
    zeros = jnp.zeros((16,), jnp.int32)
    bsl = pl.ds(base, C)

    def zero_hist():
        @pl.loop(0, NG)
        def _(i):
            hist_v[pl.ds(i * 16, 16)] = zeros

    def ingest():
        pltpu.sync_copy(times_h.at[bsl], tbuf_v)
        cp_t = pltpu.async_copy(times_h.at[bsl], t_sh.at[bsl], sem)
        cp_l = pltpu.async_copy(labels_h.at[bsl], l_sh.at[bsl], sem)
        zero_hist()
        for q in range(8):
            ones128_v[pl.ds(q * 16, 16)] = ones

        @pl.loop(0, NG)
        def _(g):
            zero256_v[pl.ds(g * 16, 16)] = zeros
        rsl = pl.ds(wid * RADIX, RADIX)
        pltpu.sync_copy(zero256_v, h21_sh.at[rsl])
        pltpu.sync_copy(zero256_v, h22_sh.at[rsl])
        pltpu.sync_copy(zero256_v, h23_sh.at[rsl])

        @pl.loop(0, NV)
        def _(i):
            x = tbuf_v[pl.ds(i * 16, 16)] + np.float32(0.0)
            k = _flip(lax.bitcast_convert_type(x, jnp.int32))
            key_v[pl.ds(i * 16, 16)] = k
            idx_v[pl.ds(i * 16, 16)] = base + i * 16 + lane

        cp_t.wait()
        cp_l.wait()

    def drain(n):
        for _ in range(n):
            pltpu.make_async_copy(times_h.at[pl.ds(0, 128)], drain_v, sem).wait()

    def pass_a0():
        @pl.loop(0, NV)
        def _(i):
            k = key_v[pl.ds(i * 16, 16)]
            d = _digit(k, 0)
            cnt, last = plsc.scan_count(d)
            plsc.addupdate_scatter(hist_v, [d], cnt, mask=last)

        pltpu.sync_copy(hist_v, hist_sh.at[pl.ds(wid * RADIX, RADIX)])

    def pass_c(p, src_k, src_i, dst_k, dst_i):
        shift = SHIFTS[p]
        cp_k = cp_i = None
        if p == 1:
            cp_k = pltpu.async_copy(src_k.at[bsl], key_v, sem2)
            cp_i = pltpu.async_copy(src_i.at[bsl], idx_v, sem2)
        elif p > 1:
            cp_k = pltpu.async_copy(src_k.at[bsl], key_v, sem2)

        src_hist = (hist_sh, h21_sh, h22_sh, h23_sh)[p]
        pltpu.sync_copy(src_hist, allh_v)

        @pl.loop(0, NG)
        def _(g):
            acc = zeros
            tp = zeros
            for t in range(NT):
                v = allh_v[pl.ds(t * RADIX + g * 16, 16)]
                tp = tp + jnp.where(t < wid, v, 0)
                acc = acc + v
            gtot_v[pl.ds(g * 16, 16)] = acc
            tpre_v[pl.ds(g * 16, 16)] = tp

        @pl.loop(0, NG)
        def _(g):
            pfx_v[pl.ds(g * 16, 16)] = plsc.cumsum(gtot_v[pl.ds(g * 16, 16)])
        tot = plsc.load_gather(pfx_v, [lane * 16 + 15])
        carry_v[...] = plsc.cumsum(tot) - tot

        @pl.loop(0, NG)
        def _(g):
            sl16 = pl.ds(g * 16, 16)
            gv = jnp.full((16,), g, jnp.int32)
            tab_v[sl16] = (pfx_v[sl16] - gtot_v[sl16] + tpre_v[sl16]
                           + plsc.load_gather(carry_v, [gv]))

        if cp_k is not None:
            cp_k.wait()
        if cp_i is not None:
            cp_i.wait()

        nxt = (h21_sh, h22_sh, h23_sh, None)[p]
        @pl.loop(0, NV // 8)
        def _(j):
            for q in range(8):
                i = j * 8 + q
                sl16 = pl.ds(i * 16, 16)
                k = key_v[sl16]
                d = _digit(k, shift)
                cnt, last = plsc.scan_count(d)
                r = plsc.load_gather(tab_v, [d]) + cnt - ones
                plsc.addupdate_scatter(tab_v, [d], cnt, mask=last)
                dest_v[j, pl.ds(q * 16, 16)] = r
                if p == 0:
                    dn = _digit(k, 8)
                elif p == 1:
                    w = lax.bitwise_or(
                        lax.bitwise_and(k, np.int32(-65536)), idx_v[sl16])
                    wv[sl16] = w
                    dn = _digit(w, 16)
                elif p == 2:
                    dn = _digit(k, 24)
                else:
                    wv[sl16] = lax.bitwise_and(k, np.int32(0x7FFF))
                if p < 3:
                    a2 = lax.bitwise_or(
                        lax.shift_left(lax.shift_right_logical(r, 11), 8), dn)
                    addr2_v[j, pl.ds(q * 16, 16)] = a2
            sl = pl.ds(j * 128, 128)
            if p == 0:
                pltpu.async_copy(key_v.at[sl], dst_k.at[dest_v.at[j]], sem)
                pltpu.async_copy(idx_v.at[sl], dst_i.at[dest_v.at[j]], sem)
            elif p == 2:
                pltpu.async_copy(key_v.at[sl], dst_k.at[dest_v.at[j]], sem)
            else:
                pltpu.async_copy(wv.at[sl], dst_k.at[dest_v.at[j]], sem)
            if p < 3:
                pltpu.async_copy(
                    ones128_v, nxt.at[addr2_v.at[j]], sem, add=True)

        drain((48, 32, 32, 16)[p])

    def final_out():
        pltpu.sync_copy(ia.at[bsl], idx_v)
        for j in range(16):
            sl = pl.ds(j * 128, 128)
            pltpu.async_copy(t_sh.at[idx_v.at[sl]], tbuf_v.at[sl], sem)
            pltpu.async_copy(l_sh.at[idx_v.at[sl]], lbuf_v.at[sl], sem)
        drain(32)
        pltpu.sync_copy(tbuf_v, out_t_h.at[bsl])
        pltpu.sync_copy(lbuf_v, out_l_h.at[bsl])

    sched = [
        (None, None, kb, ib, True),
        (kb, ib, ka, None, True),
        (ka, None, kb, None, True),
        (kb, None, ia, None, False),
    ]

    for p, (sk, si, dk, di, wk) in enumerate(sched):
        if p == 0:
            ingest()
            pass_a0()
            plsc.subcore_barrier()
        pass_c(p, sk, si, dk, di)
        plsc.subcore_barrier()

    final_out()


def _build():
    mesh = plsc.VectorSubcoreMesh(
        core_axis_name="core", subcore_axis_name="sub", num_cores=1)
    return pl.kernel(
        _sort_body,
        out_type=(jax.ShapeDtypeStruct((N,), jnp.float32),
                  jax.ShapeDtypeStruct((N,), jnp.int32)),
        mesh=mesh,
        compiler_params=pltpu.CompilerParams(needs_layout_passes=False),
        scratch_types=[
            pltpu.VMEM((C,), jnp.int32),
            pltpu.VMEM((C,), jnp.int32),
            pltpu.VMEM((RADIX,), jnp.int32),
            pltpu.VMEM((RADIX,), jnp.int32),
            pltpu.VMEM((NT * RADIX,), jnp.int32),
            pltpu.VMEM((RADIX,), jnp.int32),
            pltpu.VMEM((RADIX,), jnp.int32),
            pltpu.VMEM((RADIX,), jnp.int32),
            pltpu.VMEM((16,), jnp.int32),
            pltpu.VMEM((16, 128), jnp.int32),
            pltpu.VMEM((C,), jnp.float32),
            pltpu.VMEM((C,), jnp.int32),
            pltpu.VMEM((C,), jnp.int32),
            pltpu.VMEM((128,), jnp.int32),
            pltpu.VMEM((128,), jnp.int32),
            pltpu.VMEM((RADIX,), jnp.int32),
            pltpu.VMEM((16, 128), jnp.int32),
            pltpu.VMEM_SHARED((N,), jnp.int32),
            pltpu.VMEM_SHARED((N,), jnp.int32),
            pltpu.VMEM_SHARED((N,), jnp.int32),
            pltpu.VMEM_SHARED((N,), jnp.int32),
            pltpu.VMEM_SHARED((N,), jnp.float32),
            pltpu.VMEM_SHARED((N,), jnp.int32),
            pltpu.VMEM_SHARED((NT * RADIX,), jnp.int32),
            pltpu.VMEM_SHARED((NT * RADIX,), jnp.int32),
            pltpu.VMEM_SHARED((NT * RADIX,), jnp.int32),
            pltpu.VMEM_SHARED((NT * RADIX,), jnp.int32),
            pltpu.SemaphoreType.DMA,
            pltpu.SemaphoreType.DMA,
        ],
    )


def kernel(times, labels):
    labels32 = labels.astype(jnp.int32)
    out_t, out_l = _build()(times, labels32)
    return out_t, out_l.astype(labels.dtype)


# speedup vs baseline: 1.1187x; 1.0122x over previous
import numpy as np

import jax
import jax.numpy as jnp
from jax import lax
from jax.experimental import pallas as pl
from jax.experimental.pallas import tpu as pltpu
from jax.experimental.pallas import tpu_sc as plsc

N = 32768
NT = 16
C = N // NT
NV = C // 16
RADIX = 256
NG = RADIX // 16
SHIFTS = (0, 8, 16, 24)

_MIN32 = np.int32(-2147483648)


def _flip(v):
    m = lax.shift_right_arithmetic(v, 31)
    return lax.bitwise_xor(v, lax.bitwise_or(m, _MIN32))


def _digit(k, shift):
    return lax.bitwise_and(lax.shift_right_logical(k, shift), np.int32(255))


def _sort_body(times_h, labels_h, out_t_h, out_l_h,
               key_v, idx_v, hist_v, tab_v, allh_v, gtot_v, tpre_v,
               pfx_v, carry_v, dest_v, tbuf_v, lbuf_v, wv, drain_v,
               ones128_v, zero256_v, addr2_v,
               ka, kb, ia, ib, t_sh, l_sh, hist_sh, h21_sh, h22_sh, h23_sh,
               sem, sem2):
    wid = lax.axis_index("sub")
    base = wid * C
    lane = lax.iota(jnp.int32, 16)
    ones = jnp.ones((16,), jnp.int32)
    zeros = jnp.zeros((16,), jnp.int32)
    bsl = pl.ds(base, C)

    def zero_hist():
        @pl.loop(0, NG)
        def _(i):
            hist_v[pl.ds(i * 16, 16)] = zeros

    def ingest():
        pltpu.sync_copy(times_h.at[bsl], tbuf_v)
        cp_t = pltpu.async_copy(times_h.at[bsl], t_sh.at[bsl], sem)
        cp_l = pltpu.async_copy(labels_h.at[bsl], l_sh.at[bsl], sem)
        zero_hist()
        for q in range(8):
            ones128_v[pl.ds(q * 16, 16)] = ones

        @pl.loop(0, NG)
        def _(g):
            zero256_v[pl.ds(g * 16, 16)] = zeros
        rsl = pl.ds(wid * RADIX, RADIX)
        pltpu.sync_copy(zero256_v, h21_sh.at[rsl])
        pltpu.sync_copy(zero256_v, h22_sh.at[rsl])
        pltpu.sync_copy(zero256_v, h23_sh.at[rsl])

        @pl.loop(0, NV)
        def _(i):
            x = tbuf_v[pl.ds(i * 16, 16)] + np.float32(0.0)
            k = _flip(lax.bitcast_convert_type(x, jnp.int32))
            key_v[pl.ds(i * 16, 16)] = k
            idx_v[pl.ds(i * 16, 16)] = base + i * 16 + lane
            d = _digit(k, 0)
            cnt, last = plsc.scan_count(d)
            plsc.addupdate_scatter(hist_v, [d], cnt, mask=last)

        pltpu.sync_copy(hist_v, hist_sh.at[pl.ds(wid * RADIX, RADIX)])
        cp_t.wait()
        cp_l.wait()

    def drain(n):
        for _ in range(n):
            pltpu.make_async_copy(times_h.at[pl.ds(0, 128)], drain_v, sem).wait()

    def pass_c(p, src_k, src_i, dst_k, dst_i):
        shift = SHIFTS[p]
        cp_k = cp_i = None
        if p == 1:
            cp_k = pltpu.async_copy(src_k.at[bsl], key_v, sem2)
            cp_i = pltpu.async_copy(src_i.at[bsl], idx_v, sem2)
        elif p > 1:
            cp_k = pltpu.async_copy(src_k.at[bsl], key_v, sem2)

        src_hist = (hist_sh, h21_sh, h22_sh, h23_sh)[p]
        pltpu.sync_copy(src_hist, allh_v)

        @pl.loop(0, NG)
        def _(g):
            acc = zeros
            tp = zeros
            for t in range(NT):
                v = allh_v[pl.ds(t * RADIX + g * 16, 16)]
                tp = tp + jnp.where(t < wid, v, 0)
                acc = acc + v
            gtot_v[pl.ds(g * 16, 16)] = acc
            tpre_v[pl.ds(g * 16, 16)] = tp

        @pl.loop(0, NG)
        def _(g):
            pfx_v[pl.ds(g * 16, 16)] = plsc.cumsum(gtot_v[pl.ds(g * 16, 16)])
        tot = plsc.load_gather(pfx_v, [lane * 16 + 15])
        carry_v[...] = plsc.cumsum(tot) - tot

        @pl.loop(0, NG)
        def _(g):
            sl16 = pl.ds(g * 16, 16)
            gv = jnp.full((16,), g, jnp.int32)
            tab_v[sl16] = (pfx_v[sl16] - gtot_v[sl16] + tpre_v[sl16]
                           + plsc.load_gather(carry_v, [gv]))

        if cp_k is not None:
            cp_k.wait()
        if cp_i is not None:
            cp_i.wait()

        nxt = (h21_sh, h22_sh, h23_sh, None)[p]
        @pl.loop(0, NV // 8)
        def _(j):
            for q in range(8):
                i = j * 8 + q
                sl16 = pl.ds(i * 16, 16)
                k = key_v[sl16]
                d = _digit(k, shift)
                cnt, last = plsc.scan_count(d)
                r = plsc.load_gather(tab_v, [d]) + cnt - ones
                plsc.addupdate_scatter(tab_v, [d], cnt, mask=last)
                dest_v[j, pl.ds(q * 16, 16)] = r
                if p == 0:
                    dn = _digit(k, 8)
                elif p == 1:
                    w = lax.bitwise_or(
                        lax.bitwise_and(k, np.int32(-65536)), idx_v[sl16])
                    wv[sl16] = w
                    dn = _digit(w, 16)
                elif p == 2:
                    dn = _digit(k, 24)
                else:
                    wv[sl16] = lax.bitwise_and(k, np.int32(0x7FFF))
                if p < 3:
                    a2 = lax.bitwise_or(
                        lax.shift_left(lax.shift_right_logical(r, 11), 8), dn)
                    addr2_v[j, pl.ds(q * 16, 16)] = a2
            sl = pl.ds(j * 128, 128)
            if p == 0:
                pltpu.async_copy(key_v.at[sl], dst_k.at[dest_v.at[j]], sem)
                pltpu.async_copy(idx_v.at[sl], dst_i.at[dest_v.at[j]], sem)
            elif p == 2:
                pltpu.async_copy(key_v.at[sl], dst_k.at[dest_v.at[j]], sem)
            else:
                pltpu.async_copy(wv.at[sl], dst_k.at[dest_v.at[j]], sem)
            if p < 3:
                pltpu.async_copy(
                    ones128_v, nxt.at[addr2_v.at[j]], sem, add=True)

        drain((48, 32, 32, 16)[p])

    def final_out():
        pltpu.sync_copy(ia.at[bsl], idx_v)
        for j in range(16):
            sl = pl.ds(j * 128, 128)
            pltpu.async_copy(t_sh.at[idx_v.at[sl]], tbuf_v.at[sl], sem)
            pltpu.async_copy(l_sh.at[idx_v.at[sl]], lbuf_v.at[sl], sem)
        drain(32)
        cp_t = pltpu.async_copy(tbuf_v, out_t_h.at[bsl], sem2)
        cp_l = pltpu.async_copy(lbuf_v, out_l_h.at[bsl], sem2)
        cp_t.wait()
        cp_l.wait()

    sched = [
        (None, None, kb, ib, True),
        (kb, ib, ka, None, True),
        (ka, None, kb, None, True),
        (kb, None, ia, None, False),
    ]

    for p, (sk, si, dk, di, wk) in enumerate(sched):
        if p == 0:
            ingest()
            plsc.subcore_barrier()
        pass_c(p, sk, si, dk, di)
        plsc.subcore_barrier()

    final_out()


def _build():
    mesh = plsc.VectorSubcoreMesh(
        core_axis_name="core", subcore_axis_name="sub", num_cores=1)
    return pl.kernel(
        _sort_body,
        out_type=(jax.ShapeDtypeStruct((N,), jnp.float32),
                  jax.ShapeDtypeStruct((N,), jnp.int32)),
        mesh=mesh,
        compiler_params=pltpu.CompilerParams(needs_layout_passes=False),
        scratch_types=[
            pltpu.VMEM((C,), jnp.int32),
            pltpu.VMEM((C,), jnp.int32),
            pltpu.VMEM((RADIX,), jnp.int32),
            pltpu.VMEM((RADIX,), jnp.int32),
            pltpu.VMEM((NT * RADIX,), jnp.int32),
            pltpu.VMEM((RADIX,), jnp.int32),
            pltpu.VMEM((RADIX,), jnp.int32),
            pltpu.VMEM((RADIX,), jnp.int32),
            pltpu.VMEM((16,), jnp.int32),
            pltpu.VMEM((16, 128), jnp.int32),
            pltpu.VMEM((C,), jnp.float32),
            pltpu.VMEM((C,), jnp.int32),
            pltpu.VMEM((C,), jnp.int32),
            pltpu.VMEM((128,), jnp.int32),
            pltpu.VMEM((128,), jnp.int32),
            pltpu.VMEM((RADIX,), jnp.int32),
            pltpu.VMEM((16, 128), jnp.int32),
            pltpu.VMEM_SHARED((N,), jnp.int32),
            pltpu.VMEM_SHARED((N,), jnp.int32),
            pltpu.VMEM_SHARED((N,), jnp.int32),
            pltpu.VMEM_SHARED((N,), jnp.int32),
            pltpu.VMEM_SHARED((N,), jnp.float32),
            pltpu.VMEM_SHARED((N,), jnp.int32),
            pltpu.VMEM_SHARED((NT * RADIX,), jnp.int32),
            pltpu.VMEM_SHARED((NT * RADIX,), jnp.int32),
            pltpu.VMEM_SHARED((NT * RADIX,), jnp.int32),
            pltpu.VMEM_SHARED((NT * RADIX,), jnp.int32),
            pltpu.SemaphoreType.DMA,
            pltpu.SemaphoreType.DMA,
        ],
    )


def kernel(times, labels):
    labels32 = labels.astype(jnp.int32)
    out_t, out_l = _build()(times, labels32)
    return out_t, out_l.astype(labels.dtype)
